# baseline (speedup 1.0000x reference)
import functools

import jax
import jax.numpy as jnp
from jax import lax
from jax.experimental import pallas as pl
from jax.experimental.pallas import tpu as pltpu
from jax.experimental.pallas import tpu_sc as plsc

L = 16
NC = 2
NS = 16
NW = NC * NS
B = 1024
XM = 9.0
SCALE = B / (2.0 * XM)
STRIDE = 2 * B + 1
R = (2 * B) // L
R0 = B // L
P = 16 * 512 * 512
PER_W = P // NW
CR = 16
C = CR * 512
NCHUNK = PER_W // C
NSLOT = 4
U = 8
BSH = B.bit_length() - 1


@functools.cache
def _build():
  mesh = plsc.VectorSubcoreMesh(
      core_axis_name="c", subcore_axis_name="s", num_cores=NC, num_subcores=NS)

  @functools.partial(
      pl.kernel,
      out_type=jax.ShapeDtypeStruct((NW, R * L), jnp.float32),
      mesh=mesh,
      scratch_types=[
          [pltpu.VMEM((CR, 512), jnp.float32) for _ in range(NSLOT)],
          [pltpu.VMEM((CR, 512), jnp.int32) for _ in range(NSLOT)],
          pltpu.VMEM((L * STRIDE + L,), jnp.float32),
          pltpu.VMEM((R * L,), jnp.float32),
          [pltpu.SemaphoreType.DMA for _ in range(NSLOT)],
      ],
      compiler_params=pltpu.CompilerParams(needs_layout_passes=False),
  )
  def stage1(x_hbm, t_hbm, out_hbm, xbs, tbs, hist, red, sems):
    cid = lax.axis_index("c")
    sid = lax.axis_index("s")
    wid = sid * NC + cid
    img = lax.div(wid, 2)
    half = lax.rem(wid, 2)
    lane = lax.iota(jnp.int32, 16)
    lane_off = lane * STRIDE
    ones = jnp.ones((L,), jnp.float32)
    zeros = jnp.zeros((L,), jnp.float32)

    def zh(i, _):
      for u in range(8):
        hist[pl.ds((i * 8 + u) * L, L)] = zeros
      return 0
    lax.fori_loop(0, (L * STRIDE + L - 1) // (8 * L) + 1, zh, 0)

    bufs = [(xbs[s], tbs[s], sems[s]) for s in range(NSLOT)]

    def issue(c, slot):
      xb, tb, sem = bufs[slot]
      rb = half * 256 + c * CR
      cx = pltpu.async_copy(x_hbm.at[img, 0, pl.ds(rb, CR), :], xb, sem)
      ct = pltpu.async_copy(t_hbm.at[img, 0, pl.ds(rb, CR), :], tb, sem)
      return cx, ct

    XMU = XM - 1.5 / SCALE

    def compute(slot):
      xb, tb, _ = bufs[slot]

      def body(r, _):
        for k0 in range(0, 512 // L, U):
          xs = [xb[r, pl.ds((k0 + u) * L, L)] for u in range(U)]
          ts = [tb[r, pl.ds((k0 + u) * L, L)] for u in range(U)]
          idxs = []
          for u in range(U):
            uv = jnp.minimum(jnp.maximum(xs[u], -XM), XMU)
            bi = ((uv + XM) * SCALE).astype(jnp.int32)
            idxs.append(lane_off + bi + jnp.left_shift(ts[u], BSH))
          for u in range(U):
            plsc.addupdate_scatter(hist, [idxs[u]], ones)
        return 0
      lax.fori_loop(0, CR, body, 0)

    pending = {}
    for s in range(min(NSLOT, NCHUNK)):
      pending[s] = issue(s, s)
    for c in range(NCHUNK):
      slot = c % NSLOT
      cpx, cpt = pending[slot]
      cpx.wait()
      cpt.wait()
      compute(slot)
      if c + NSLOT < NCHUNK:
        pending[slot] = issue(c + NSLOT, slot)

    def red_body(j, _):
      base = j * L + lane
      vals = [plsc.load_gather(hist, [base + l * STRIDE]) for l in range(L)]
      while len(vals) > 1:
        vals = [vals[k] + vals[k + 1] for k in range(0, len(vals), 2)]
      red[pl.ds(j * L, L)] = vals[0]
      return 0
    lax.fori_loop(0, R, red_body, 0)

    pltpu.sync_copy(red, out_hbm.at[wid])

  RT = R // NS

  @functools.partial(
      pl.kernel,
      out_type=jax.ShapeDtypeStruct((L,), jnp.float32),
      mesh=mesh,
      scratch_types=[
          pltpu.VMEM((NW, RT * L), jnp.float32),
          pltpu.VMEM((RT * L,), jnp.float32),
          pltpu.VMEM((R * L,), jnp.float32),
          pltpu.VMEM((L,), jnp.float32),
          pltpu.VMEM_SHARED((R * L,), jnp.float32),
          pltpu.SemaphoreType.DMA,
      ],
      compiler_params=pltpu.CompilerParams(needs_layout_passes=False),
  )
  def stage2(part_hbm, out_hbm, blk, red16, acc, obuf, shared, sem):
    cid = lax.axis_index("c")
    sid = lax.axis_index("s")

    @pl.when(cid == 0)
    def _():
      pltpu.async_copy(
          part_hbm.at[:, pl.ds(sid * RT * L, RT * L)], blk, sem).wait()
      for r in range(RT):
        vals = [blk[j, pl.ds(r * L, L)] for j in range(NW)]
        while len(vals) > 1:
          vals = [vals[k] + vals[k + 1] for k in range(0, len(vals), 2)]
        red16[pl.ds(r * L, L)] = vals[0]
      pltpu.sync_copy(red16, shared.at[pl.ds(sid * RT * L, RT * L)])
    plsc.subcore_barrier()

    @pl.when((sid == 0) & (cid == 0))
    def _():
      lane = lax.iota(jnp.int32, 16)
      lanef = lane.astype(jnp.float32)
      zeros = jnp.zeros((L,), jnp.float32)
      pltpu.sync_copy(shared, acc)

      def sig_of(midx):
        e = jnp.exp(-jnp.abs(midx))
        r = 1.0 / (1.0 + e)
        sm = e * r
        pos = midx >= 0.0
        return jnp.where(pos, r, sm), jnp.where(pos, sm, r)

      def l1_body(j, carry):
        g_acc, s1_acc = carry
        n1 = acc[pl.ds((R0 + j) * L, L)]
        midx = (j * L + lanef + 0.5) * (1.0 / SCALE) - XM
        _, omp = sig_of(midx)
        return g_acc + n1, s1_acc + n1 * omp
      g_acc, s1_acc = lax.fori_loop(0, R0, l1_body, (zeros, zeros))
      g = jnp.sum(g_acc)
      s1 = jnp.sum(s1_acc)
      gs = jnp.maximum(g, 1.0)

      def l0_body(j, carry):
        run, tsum, topsig = carry
        rj = R0 - 1 - j
        nrev = lax.rev(acc[pl.ds(rj * L, L)], (0,))
        r1v = run + plsc.cumsum(nrev)
        r0v = r1v - nrev
        midx = (rj * L + (float(L - 1) - lanef) + 0.5) * (1.0 / SCALE) - XM
        sig, _ = sig_of(midx)
        gvec = zeros + g
        w = gvec / ((gs + r0v) * (gs + r1v))
        tsum = tsum + nrev * (1.0 + sig) * w
        run = run + jnp.sum(nrev)
        topsig = jnp.maximum(topsig, jnp.max(jnp.where(nrev > 0.0, sig, -1.0)))
        return run, tsum, topsig
      _, tsum, topsig = lax.fori_loop(
          0, R0, l0_body, (zeros, zeros, -1.0))

      loss = s1 * (1.0 / float(P)) + jnp.sum(tsum)
      loss = jnp.where(g > 0.0, loss, 1.0 + topsig)
      obuf[...] = zeros + loss
      pltpu.sync_copy(obuf, out_hbm)

  return stage1, stage2


def kernel(inputs, targets):
  stage1, stage2 = _build()
  partials = stage1(inputs, targets)
  out = stage2(partials)
  return out[0]


# speedup vs baseline: 1.0183x; 1.0183x over previous
import functools

import jax
import jax.numpy as jnp
from jax import lax
from jax.experimental import pallas as pl
from jax.experimental.pallas import tpu as pltpu
from jax.experimental.pallas import tpu_sc as plsc

L = 16
NC = 2
NS = 16
NW = NC * NS
B = 1024
XM = 9.0
SCALE = B / (2.0 * XM)
STRIDE = 2 * B + 1
R = (2 * B) // L
R0 = B // L
P = 16 * 512 * 512
PER_W = P // NW
CR = 16
C = CR * 512
NCHUNK = PER_W // C
NSLOT = 2
U = 8
BSH = B.bit_length() - 1


@functools.cache
def _build():
  mesh = plsc.VectorSubcoreMesh(
      core_axis_name="c", subcore_axis_name="s", num_cores=NC, num_subcores=NS)

  @functools.partial(
      pl.kernel,
      out_type=jax.ShapeDtypeStruct((NW, R * L), jnp.float32),
      mesh=mesh,
      scratch_types=[
          [pltpu.VMEM((CR, 512), jnp.float32) for _ in range(NSLOT)],
          [pltpu.VMEM((CR, 512), jnp.int32) for _ in range(NSLOT)],
          pltpu.VMEM((L * STRIDE + L,), jnp.float32),
          pltpu.VMEM((R * L,), jnp.float32),
          [pltpu.SemaphoreType.DMA for _ in range(NSLOT)],
      ],
      compiler_params=pltpu.CompilerParams(needs_layout_passes=False),
  )
  def stage1(x_hbm, t_hbm, out_hbm, xbs, tbs, hist, red, sems):
    cid = lax.axis_index("c")
    sid = lax.axis_index("s")
    wid = sid * NC + cid
    img = lax.div(wid, 2)
    half = lax.rem(wid, 2)
    lane = lax.iota(jnp.int32, 16)
    lane_off = lane * STRIDE
    ones = jnp.ones((L,), jnp.float32)
    zeros = jnp.zeros((L,), jnp.float32)

    def zh(i, _):
      for u in range(8):
        hist[pl.ds((i * 8 + u) * L, L)] = zeros
      return 0
    lax.fori_loop(0, (L * STRIDE + L - 1) // (8 * L) + 1, zh, 0)

    bufs = [(xbs[s], tbs[s], sems[s]) for s in range(NSLOT)]

    def issue(c, slot):
      xb, tb, sem = bufs[slot]
      rb = half * 256 + c * CR
      cx = pltpu.async_copy(x_hbm.at[img, 0, pl.ds(rb, CR), :], xb, sem)
      ct = pltpu.async_copy(t_hbm.at[img, 0, pl.ds(rb, CR), :], tb, sem)
      return cx, ct

    XMU = XM - 1.5 / SCALE

    def compute(slot):
      xb, tb, _ = bufs[slot]

      def body(r, _):
        for k0 in range(0, 512 // L, U):
          xs = [xb[r, pl.ds((k0 + u) * L, L)] for u in range(U)]
          ts = [tb[r, pl.ds((k0 + u) * L, L)] for u in range(U)]
          idxs = []
          for u in range(U):
            uv = jnp.minimum(jnp.maximum(xs[u], -XM), XMU)
            bi = ((uv + XM) * SCALE).astype(jnp.int32)
            idxs.append(lane_off + bi + jnp.left_shift(ts[u], BSH))
          for u in range(U):
            plsc.addupdate_scatter(hist, [idxs[u]], ones)
        return 0
      lax.fori_loop(0, CR, body, 0)

    pending = {}
    for s in range(min(NSLOT, NCHUNK)):
      pending[s] = issue(s, s)
    for c in range(NCHUNK):
      slot = c % NSLOT
      cpx, cpt = pending[slot]
      cpx.wait()
      cpt.wait()
      compute(slot)
      if c + NSLOT < NCHUNK:
        pending[slot] = issue(c + NSLOT, slot)

    def red_body(j, _):
      base = j * L + lane
      vals = [plsc.load_gather(hist, [base + l * STRIDE]) for l in range(L)]
      while len(vals) > 1:
        vals = [vals[k] + vals[k + 1] for k in range(0, len(vals), 2)]
      red[pl.ds(j * L, L)] = vals[0]
      return 0
    lax.fori_loop(0, R, red_body, 0)

    pltpu.sync_copy(red, out_hbm.at[wid])

  RT = R // NS

  @functools.partial(
      pl.kernel,
      out_type=jax.ShapeDtypeStruct((L,), jnp.float32),
      mesh=mesh,
      scratch_types=[
          pltpu.VMEM((NW, RT * L), jnp.float32),
          pltpu.VMEM((RT * L,), jnp.float32),
          pltpu.VMEM((R * L,), jnp.float32),
          pltpu.VMEM((L,), jnp.float32),
          pltpu.VMEM_SHARED((R * L,), jnp.float32),
          pltpu.SemaphoreType.DMA,
      ],
      compiler_params=pltpu.CompilerParams(needs_layout_passes=False),
  )
  def stage2(part_hbm, out_hbm, blk, red16, acc, obuf, shared, sem):
    cid = lax.axis_index("c")
    sid = lax.axis_index("s")

    @pl.when(cid == 0)
    def _():
      pltpu.async_copy(
          part_hbm.at[:, pl.ds(sid * RT * L, RT * L)], blk, sem).wait()
      for r in range(RT):
        vals = [blk[j, pl.ds(r * L, L)] for j in range(NW)]
        while len(vals) > 1:
          vals = [vals[k] + vals[k + 1] for k in range(0, len(vals), 2)]
        red16[pl.ds(r * L, L)] = vals[0]
      pltpu.sync_copy(red16, shared.at[pl.ds(sid * RT * L, RT * L)])
    plsc.subcore_barrier()

    @pl.when((sid == 0) & (cid == 0))
    def _():
      lane = lax.iota(jnp.int32, 16)
      lanef = lane.astype(jnp.float32)
      zeros = jnp.zeros((L,), jnp.float32)
      pltpu.sync_copy(shared, acc)

      def sig_of(midx):
        e = jnp.exp(-jnp.abs(midx))
        r = 1.0 / (1.0 + e)
        sm = e * r
        pos = midx >= 0.0
        return jnp.where(pos, r, sm), jnp.where(pos, sm, r)

      def l1_body(j, carry):
        g_acc, s1_acc = carry
        n1 = acc[pl.ds((R0 + j) * L, L)]
        midx = (j * L + lanef + 0.5) * (1.0 / SCALE) - XM
        _, omp = sig_of(midx)
        return g_acc + n1, s1_acc + n1 * omp
      g_acc, s1_acc = lax.fori_loop(0, R0, l1_body, (zeros, zeros))
      g = jnp.sum(g_acc)
      s1 = jnp.sum(s1_acc)
      gs = jnp.maximum(g, 1.0)

      def l0_body(j, carry):
        run, tsum, topsig = carry
        rj = R0 - 1 - j
        nrev = lax.rev(acc[pl.ds(rj * L, L)], (0,))
        r1v = run + plsc.cumsum(nrev)
        r0v = r1v - nrev
        midx = (rj * L + (float(L - 1) - lanef) + 0.5) * (1.0 / SCALE) - XM
        sig, _ = sig_of(midx)
        gvec = zeros + g
        w = gvec / ((gs + r0v) * (gs + r1v))
        tsum = tsum + nrev * (1.0 + sig) * w
        run = run + jnp.sum(nrev)
        topsig = jnp.maximum(topsig, jnp.max(jnp.where(nrev > 0.0, sig, -1.0)))
        return run, tsum, topsig
      _, tsum, topsig = lax.fori_loop(
          0, R0, l0_body, (zeros, zeros, -1.0))

      loss = s1 * (1.0 / float(P)) + jnp.sum(tsum)
      loss = jnp.where(g > 0.0, loss, 1.0 + topsig)
      obuf[...] = zeros + loss
      pltpu.sync_copy(obuf, out_hbm)

  return stage1, stage2


def kernel(inputs, targets):
  stage1, stage2 = _build()
  partials = stage1(inputs, targets)
  out = stage2(partials)
  return out[0]


# speedup vs baseline: 1.0185x; 1.0002x over previous
import functools

import jax
import jax.numpy as jnp
from jax import lax
from jax.experimental import pallas as pl
from jax.experimental.pallas import tpu as pltpu
from jax.experimental.pallas import tpu_sc as plsc

L = 16
NC = 2
NS = 16
NW = NC * NS
B = 1024
XM = 9.0
SCALE = B / (2.0 * XM)
STRIDE = 2 * B + 1
R = (2 * B) // L
HW = -(-(L * STRIDE + L) // 128) * 128
R0 = B // L
P = 16 * 512 * 512
PER_W = P // NW
CR = 16
C = CR * 512
NCHUNK = PER_W // C
NSLOT = 2
U = 8
BSH = B.bit_length() - 1


@functools.cache
def _build():
  mesh = plsc.VectorSubcoreMesh(
      core_axis_name="c", subcore_axis_name="s", num_cores=NC, num_subcores=NS)

  @functools.partial(
      pl.kernel,
      out_type=jax.ShapeDtypeStruct((NW, R * L), jnp.float32),
      mesh=mesh,
      scratch_types=[
          [pltpu.VMEM((CR, 512), jnp.float32) for _ in range(NSLOT)],
          [pltpu.VMEM((CR, 512), jnp.int32) for _ in range(NSLOT)],
          pltpu.VMEM((HW,), jnp.float32),
          pltpu.VMEM((R * L,), jnp.float32),
          [pltpu.SemaphoreType.DMA for _ in range(NSLOT)],
      ],
      compiler_params=pltpu.CompilerParams(needs_layout_passes=False),
  )
  def stage1(x_hbm, t_hbm, out_hbm, xbs, tbs, hist, red, sems):
    cid = lax.axis_index("c")
    sid = lax.axis_index("s")
    wid = sid * NC + cid
    img = lax.div(wid, 2)
    half = lax.rem(wid, 2)
    lane = lax.iota(jnp.int32, 16)
    lane_off = lane * STRIDE
    ones = jnp.ones((L,), jnp.float32)
    zeros = jnp.zeros((L,), jnp.float32)

    def zh(i, _):
      for u in range(8):
        hist[pl.ds((i * 8 + u) * L, L)] = zeros
      return 0
    lax.fori_loop(0, HW // (8 * L), zh, 0)

    bufs = [(xbs[s], tbs[s], sems[s]) for s in range(NSLOT)]

    def issue(c, slot):
      xb, tb, sem = bufs[slot]
      rb = half * 256 + c * CR
      cx = pltpu.async_copy(x_hbm.at[img, 0, pl.ds(rb, CR), :], xb, sem)
      ct = pltpu.async_copy(t_hbm.at[img, 0, pl.ds(rb, CR), :], tb, sem)
      return cx, ct

    XMU = XM - 1.5 / SCALE

    def compute(slot):
      xb, tb, _ = bufs[slot]

      def body(r, _):
        for k0 in range(0, 512 // L, U):
          xs = [xb[r, pl.ds((k0 + u) * L, L)] for u in range(U)]
          ts = [tb[r, pl.ds((k0 + u) * L, L)] for u in range(U)]
          idxs = []
          for u in range(U):
            uv = jnp.minimum(jnp.maximum(xs[u], -XM), XMU)
            bi = ((uv + XM) * SCALE).astype(jnp.int32)
            idxs.append(lane_off + bi + jnp.left_shift(ts[u], BSH))
          for u in range(U):
            plsc.addupdate_scatter(hist, [idxs[u]], ones)
        return 0
      lax.fori_loop(0, CR, body, 0)

    pending = {}
    for s in range(min(NSLOT, NCHUNK)):
      pending[s] = issue(s, s)
    for c in range(NCHUNK):
      slot = c % NSLOT
      cpx, cpt = pending[slot]
      cpx.wait()
      cpt.wait()
      compute(slot)
      if c + NSLOT < NCHUNK:
        pending[slot] = issue(c + NSLOT, slot)

    def red_body(j, _):
      base = j * L + lane
      vals = [plsc.load_gather(hist, [base + l * STRIDE]) for l in range(L)]
      while len(vals) > 1:
        vals = [vals[k] + vals[k + 1] for k in range(0, len(vals), 2)]
      red[pl.ds(j * L, L)] = vals[0]
      return 0
    lax.fori_loop(0, R, red_body, 0)

    pltpu.sync_copy(red, out_hbm.at[wid])

  RT = R // NS

  @functools.partial(
      pl.kernel,
      out_type=jax.ShapeDtypeStruct((L,), jnp.float32),
      mesh=mesh,
      scratch_types=[
          pltpu.VMEM((NW, RT * L), jnp.float32),
          pltpu.VMEM((RT * L,), jnp.float32),
          pltpu.VMEM((R * L,), jnp.float32),
          pltpu.VMEM((L,), jnp.float32),
          pltpu.VMEM_SHARED((R * L,), jnp.float32),
          pltpu.SemaphoreType.DMA,
      ],
      compiler_params=pltpu.CompilerParams(needs_layout_passes=False),
  )
  def stage2(part_hbm, out_hbm, blk, red16, acc, obuf, shared, sem):
    cid = lax.axis_index("c")
    sid = lax.axis_index("s")

    @pl.when(cid == 0)
    def _():
      pltpu.async_copy(
          part_hbm.at[:, pl.ds(sid * RT * L, RT * L)], blk, sem).wait()
      for r in range(RT):
        vals = [blk[j, pl.ds(r * L, L)] for j in range(NW)]
        while len(vals) > 1:
          vals = [vals[k] + vals[k + 1] for k in range(0, len(vals), 2)]
        red16[pl.ds(r * L, L)] = vals[0]
      pltpu.sync_copy(red16, shared.at[pl.ds(sid * RT * L, RT * L)])
    plsc.subcore_barrier()

    @pl.when((sid == 0) & (cid == 0))
    def _():
      lane = lax.iota(jnp.int32, 16)
      lanef = lane.astype(jnp.float32)
      zeros = jnp.zeros((L,), jnp.float32)
      pltpu.sync_copy(shared, acc)

      def sig_of(midx):
        e = jnp.exp(-jnp.abs(midx))
        r = 1.0 / (1.0 + e)
        sm = e * r
        pos = midx >= 0.0
        return jnp.where(pos, r, sm), jnp.where(pos, sm, r)

      def l1_body(j, carry):
        g_acc, s1_acc = carry
        n1 = acc[pl.ds((R0 + j) * L, L)]
        midx = (j * L + lanef + 0.5) * (1.0 / SCALE) - XM
        _, omp = sig_of(midx)
        return g_acc + n1, s1_acc + n1 * omp
      g_acc, s1_acc = lax.fori_loop(0, R0, l1_body, (zeros, zeros))
      g = jnp.sum(g_acc)
      s1 = jnp.sum(s1_acc)
      gs = jnp.maximum(g, 1.0)

      def l0_body(j, carry):
        run, tsum, topsig = carry
        rj = R0 - 1 - j
        nrev = lax.rev(acc[pl.ds(rj * L, L)], (0,))
        r1v = run + plsc.cumsum(nrev)
        r0v = r1v - nrev
        midx = (rj * L + (float(L - 1) - lanef) + 0.5) * (1.0 / SCALE) - XM
        sig, _ = sig_of(midx)
        gvec = zeros + g
        w = gvec / ((gs + r0v) * (gs + r1v))
        tsum = tsum + nrev * (1.0 + sig) * w
        run = run + jnp.sum(nrev)
        topsig = jnp.maximum(topsig, jnp.max(jnp.where(nrev > 0.0, sig, -1.0)))
        return run, tsum, topsig
      _, tsum, topsig = lax.fori_loop(
          0, R0, l0_body, (zeros, zeros, -1.0))

      loss = s1 * (1.0 / float(P)) + jnp.sum(tsum)
      loss = jnp.where(g > 0.0, loss, 1.0 + topsig)
      obuf[...] = zeros + loss
      pltpu.sync_copy(obuf, out_hbm)

  return stage1, stage2


def kernel(inputs, targets):
  stage1, stage2 = _build()
  partials = stage1(inputs, targets)
  out = stage2(partials)
  return out[0]
